# Initial kernel scaffold; baseline (speedup 1.0000x reference)
import jax
import jax.numpy as jnp
from jax.experimental import pallas as pl


def kernel(x, edge_index, W1, att_src1, att_dst1, bias1, W2, att_src2, att_dst2, bias2):
    raise NotImplementedError("write your pallas kernel here")


# speedup vs baseline: 42.0339x; 42.0339x over previous
import functools

import jax
import jax.numpy as jnp
from jax import lax
from jax.experimental import pallas as pl
from jax.experimental.pallas import tpu as pltpu
from jax.experimental.pallas import tpu_sc as plsc

N = 10000
E = 320000
IN_CH = 128
HID = 16
HEADS = 8
OUT = 16

NPAD = 10240
ROWS_PT = NPAD // 16
C = 128
NW = 32
K = 81
EPAD = NW * C * K

f32 = jnp.float32
i32 = jnp.int32


def _dense1_body(x_ref, w1_ref, asw_ref, adw_ref,
                 h1_ref, asp_ref, adp_ref, maxb_ref):
    i = pl.program_id(0)
    h = jnp.dot(x_ref[...], w1_ref[...], preferred_element_type=f32)
    h1_ref[...] = h
    asp = jnp.dot(h, asw_ref[...], preferred_element_type=f32)
    adp = jnp.dot(h, adw_ref[...], preferred_element_type=f32)
    asp_ref[...] = asp
    adp_ref[...] = adp

    @pl.when(i == 0)
    def _():
        maxb_ref[...] = jnp.zeros_like(maxb_ref)

    ms = jnp.max(asp, axis=0)[None, :]
    md = jnp.max(adp, axis=0)[None, :]
    maxb_ref[0:1, 0:16] = jnp.maximum(maxb_ref[0:1, 0:16], ms)
    maxb_ref[1:2, 0:16] = jnp.maximum(maxb_ref[1:2, 0:16], md)


def _dense2_body(m0_ref, m1_ref, w0_ref, w1_ref, b1_ref, w2_ref, r_ref,
                 as2w_ref, ad2w_ref,
                 g2_ref, as2p_ref, ad2p_ref, maxb2_ref):
    i = pl.program_id(0)
    msg = m0_ref[...] + m1_ref[...]
    den = w0_ref[...] + w1_ref[...]
    denrep = jnp.dot(den, r_ref[...], preferred_element_type=f32)
    out1 = msg / (denrep + 1e-16) + b1_ref[...]
    h2 = jnp.where(out1 > 0.0, out1, jnp.exp(jnp.minimum(out1, 0.0)) - 1.0)
    g2 = jnp.dot(h2, w2_ref[...], preferred_element_type=f32)
    g2_ref[...] = g2
    as2p = jnp.dot(g2, as2w_ref[...], preferred_element_type=f32)
    ad2p = jnp.dot(g2, ad2w_ref[...], preferred_element_type=f32)
    as2p_ref[...] = as2p
    ad2p_ref[...] = ad2p

    @pl.when(i == 0)
    def _():
        maxb2_ref[...] = jnp.zeros_like(maxb2_ref)

    ms = jnp.max(as2p, axis=0)[None, :]
    md = jnp.max(ad2p, axis=0)[None, :]
    maxb2_ref[0:1, 0:16] = jnp.maximum(maxb2_ref[0:1, 0:16], ms)
    maxb2_ref[1:2, 0:16] = jnp.maximum(maxb2_ref[1:2, 0:16], md)


def _final_body(m0_ref, m1_ref, w0_ref, w1_ref, b2_ref, bsel_ref, out_ref):
    den = jnp.dot(w0_ref[...] + w1_ref[...], bsel_ref[...],
                  preferred_element_type=f32)
    out_ref[...] = (m0_ref[...] + m1_ref[...]) / (den + 1e-16) + b2_ref[...]


def _make_sc1():
    mesh = plsc.VectorSubcoreMesh(core_axis_name="c", subcore_axis_name="s",
                                  num_cores=2, num_subcores=16)

    @functools.partial(
        pl.kernel, mesh=mesh,
        compiler_params=pltpu.CompilerParams(needs_layout_passes=False,
                                             use_tc_tiling_on_sc=False),
        out_type=[jax.ShapeDtypeStruct((2, NPAD, IN_CH), f32),
                  jax.ShapeDtypeStruct((2, NPAD, 16), f32)],
        scratch_types=[
            pltpu.VMEM((C,), i32), pltpu.VMEM((C,), i32),
            pltpu.VMEM((C, IN_CH), f32), pltpu.VMEM((C, 16), f32),
            pltpu.VMEM((C, 16), f32), pltpu.VMEM((C, 16), f32),
            pltpu.VMEM((16,), f32), pltpu.VMEM((16,), f32),
            pltpu.VMEM_SHARED((NPAD, IN_CH), f32),
            pltpu.VMEM_SHARED((NPAD, 16), f32),
            pltpu.SemaphoreType.DMA, pltpu.SemaphoreType.DMA,
            pltpu.SemaphoreType.DMA,
        ],
    )
    def sc1(h1, asp, adp, s_idx, d_idx, maxb, zm, zw,
            outm, outw,
            sidx, didx, hrows, asr, adr, wzbuf, msv, mdv,
            accm, accw, sem1, sem2, sem3):
        cid = lax.axis_index("c")
        sid = lax.axis_index("s")
        wid = cid * 16 + sid
        r0 = sid * ROWS_PT
        pltpu.sync_copy(zm, accm.at[pl.ds(r0, ROWS_PT)])
        pltpu.sync_copy(zw, accw.at[pl.ds(r0, ROWS_PT)])
        pltpu.sync_copy(maxb.at[0, pl.ds(0, 16)], msv)
        pltpu.sync_copy(maxb.at[1, pl.ds(0, 16)], mdv)
        msum = msv[...] + mdv[...]
        msv[...] = jnp.where(msum > 0.0, msum, 0.2 * msum)

        def _zb(r, carry):
            wzbuf[r, :] = jnp.zeros((16,), f32)
            return carry
        lax.fori_loop(0, C, _zb, 0)
        plsc.subcore_barrier()

        lanes = lax.iota(i32, 16)
        mvals = msv[...]
        mh = [mvals[h] for h in range(HEADS)]
        ebase = wid * (C * K)

        def chunk(k, carry):
            base = ebase + k * C
            pltpu.sync_copy(s_idx.at[pl.ds(base, C)], sidx)
            pltpu.sync_copy(d_idx.at[pl.ds(base, C)], didx)
            cp1 = pltpu.async_copy(h1.at[sidx], hrows, sem1)
            cp2 = pltpu.async_copy(asp.at[sidx], asr, sem2)
            cp3 = pltpu.async_copy(adp.at[didx], adr, sem3)
            cp1.wait()
            cp2.wait()
            cp3.wait()
            for g in range(C // 16):
                eidx = lanes + (g * 16)
                for h in range(HEADS):
                    hsplat = jnp.full((16,), h, i32)
                    asv = plsc.load_gather(asr, [eidx, hsplat])
                    adv = plsc.load_gather(adr, [eidx, hsplat])
                    al = asv + adv
                    al = jnp.where(al > 0.0, al, 0.2 * al)
                    w = jnp.exp(al - mh[h])
                    plsc.store_scatter(wzbuf, [eidx, hsplat], w)

            def edge(e, carry2):
                esplat = lanes * 0 + e
                for h in range(HEADS):
                    hsplat = jnp.full((16,), h, i32)
                    wh = plsc.load_gather(wzbuf, [esplat, hsplat])
                    seg = hrows[e, pl.ds(h * HID, HID)]
                    hrows[e, pl.ds(h * HID, HID)] = seg * wh
                return carry2
            lax.fori_loop(0, C, edge, 0)

            pltpu.sync_copy(hrows, accm.at[didx], add=True)
            pltpu.sync_copy(wzbuf, accw.at[didx], add=True)
            return carry
        lax.fori_loop(0, K, chunk, 0)
        plsc.subcore_barrier()
        pltpu.sync_copy(accm.at[pl.ds(r0, ROWS_PT)],
                        outm.at[cid, pl.ds(r0, ROWS_PT)])
        pltpu.sync_copy(accw.at[pl.ds(r0, ROWS_PT)],
                        outw.at[cid, pl.ds(r0, ROWS_PT)])

    return sc1


def _make_sc2():
    mesh = plsc.VectorSubcoreMesh(core_axis_name="c", subcore_axis_name="s",
                                  num_cores=2, num_subcores=16)

    @functools.partial(
        pl.kernel, mesh=mesh,
        compiler_params=pltpu.CompilerParams(needs_layout_passes=False,
                                             use_tc_tiling_on_sc=False),
        out_type=[jax.ShapeDtypeStruct((2, NPAD, 16), f32),
                  jax.ShapeDtypeStruct((2, NPAD, 16), f32)],
        scratch_types=[
            pltpu.VMEM((C,), i32), pltpu.VMEM((C,), i32),
            pltpu.VMEM((C, 16), f32), pltpu.VMEM((C, 16), f32),
            pltpu.VMEM((C, 16), f32), pltpu.VMEM((C, 16), f32),
            pltpu.VMEM((16,), f32), pltpu.VMEM((16,), f32),
            pltpu.VMEM_SHARED((NPAD, 16), f32),
            pltpu.VMEM_SHARED((NPAD, 16), f32),
            pltpu.SemaphoreType.DMA, pltpu.SemaphoreType.DMA,
            pltpu.SemaphoreType.DMA,
        ],
    )
    def sc2(g2, as2p, ad2p, s_idx, d_idx, maxb2, zw,
            outm, outw,
            sidx, didx, grows, asr, adr, wzbuf, msv, mdv,
            accm, accw, sem1, sem2, sem3):
        cid = lax.axis_index("c")
        sid = lax.axis_index("s")
        wid = cid * 16 + sid
        r0 = sid * ROWS_PT
        pltpu.sync_copy(zw, accm.at[pl.ds(r0, ROWS_PT)])
        pltpu.sync_copy(zw, accw.at[pl.ds(r0, ROWS_PT)])
        pltpu.sync_copy(maxb2.at[0, pl.ds(0, 16)], msv)
        pltpu.sync_copy(maxb2.at[1, pl.ds(0, 16)], mdv)
        msum = msv[...] + mdv[...]
        mvec = jnp.where(msum > 0.0, msum, 0.2 * msum)
        plsc.subcore_barrier()

        lanes = lax.iota(i32, 16)
        lane0 = lanes == 0
        ebase = wid * (C * K)

        def chunk(k, carry):
            base = ebase + k * C
            pltpu.sync_copy(s_idx.at[pl.ds(base, C)], sidx)
            pltpu.sync_copy(d_idx.at[pl.ds(base, C)], didx)
            cp1 = pltpu.async_copy(g2.at[sidx], grows, sem1)
            cp2 = pltpu.async_copy(as2p.at[sidx], asr, sem2)
            cp3 = pltpu.async_copy(ad2p.at[didx], adr, sem3)
            cp1.wait()
            cp2.wait()
            cp3.wait()

            def edge(e, carry2):
                al = asr[e, :] + adr[e, :]
                al = jnp.where(al > 0.0, al, 0.2 * al)
                w = jnp.exp(al - mvec)
                grows[e, :] = grows[e, :] * w
                wzbuf[e, :] = jnp.where(lane0, w, 0.0)
                return carry2
            lax.fori_loop(0, C, edge, 0)

            pltpu.sync_copy(grows, accm.at[didx], add=True)
            pltpu.sync_copy(wzbuf, accw.at[didx], add=True)
            return carry
        lax.fori_loop(0, K, chunk, 0)
        plsc.subcore_barrier()
        pltpu.sync_copy(accm.at[pl.ds(r0, ROWS_PT)],
                        outm.at[cid, pl.ds(r0, ROWS_PT)])
        pltpu.sync_copy(accw.at[pl.ds(r0, ROWS_PT)],
                        outw.at[cid, pl.ds(r0, ROWS_PT)])

    return sc2


_make_sc1 = functools.cache(_make_sc1)
_make_sc2 = functools.cache(_make_sc2)

_GRID = 10
_BLK = N // _GRID


def _full(shape):
    return pl.BlockSpec(shape, lambda *_: (0,) * len(shape))


def kernel(x, edge_index, W1, att_src1, att_dst1, bias1,
           W2, att_src2, att_dst2, bias2):
    src = edge_index[0]
    dst = edge_index[1]
    loop = jnp.arange(N, dtype=i32)
    padn = EPAD - (E + N)
    s_full = jnp.concatenate([src, loop, jnp.zeros((padn,), i32)])
    d_full = jnp.concatenate([dst, loop, jnp.full((padn,), NPAD - 1, i32)])

    eye8 = jnp.eye(HEADS, dtype=f32)
    AS1 = jnp.pad((eye8[:, None, :] * att_src1[:, :, None]).reshape(IN_CH, HEADS),
                  ((0, 0), (0, 8)))
    AD1 = jnp.pad((eye8[:, None, :] * att_dst1[:, :, None]).reshape(IN_CH, HEADS),
                  ((0, 0), (0, 8)))
    R = jnp.pad(jnp.repeat(eye8, HID, axis=1), ((0, 8), (0, 0)))
    AS2 = jnp.tile(att_src2.reshape(OUT, 1), (1, 16))
    AD2 = jnp.tile(att_dst2.reshape(OUT, 1), (1, 16))
    BSEL = jnp.zeros((16, 16), f32).at[0].set(1.0)
    ZM = jnp.zeros((ROWS_PT, IN_CH), f32)
    ZW = jnp.zeros((ROWS_PT, 16), f32)

    h1, asp, adp, maxb = pl.pallas_call(
        _dense1_body,
        grid=(_GRID,),
        in_specs=[pl.BlockSpec((_BLK, IN_CH), lambda i: (i, 0)),
                  _full((IN_CH, IN_CH)), _full((IN_CH, 16)),
                  _full((IN_CH, 16))],
        out_specs=[pl.BlockSpec((_BLK, IN_CH), lambda i: (i, 0)),
                   pl.BlockSpec((_BLK, 16), lambda i: (i, 0)),
                   pl.BlockSpec((_BLK, 16), lambda i: (i, 0)),
                   _full((8, 128))],
        out_shape=[jax.ShapeDtypeStruct((N, IN_CH), f32),
                   jax.ShapeDtypeStruct((N, 16), f32),
                   jax.ShapeDtypeStruct((N, 16), f32),
                   jax.ShapeDtypeStruct((8, 128), f32)],
    )(x, W1, AS1, AD1)

    adp_p = jnp.concatenate([adp, jnp.zeros((NPAD - N, 16), f32)], axis=0)
    outm, outw = _make_sc1()(h1, asp, adp_p, s_full, d_full, maxb, ZM, ZW)

    g2, as2p, ad2p, maxb2 = pl.pallas_call(
        _dense2_body,
        grid=(_GRID,),
        in_specs=[pl.BlockSpec((_BLK, IN_CH), lambda i: (i, 0)),
                  pl.BlockSpec((_BLK, IN_CH), lambda i: (i, 0)),
                  pl.BlockSpec((_BLK, 16), lambda i: (i, 0)),
                  pl.BlockSpec((_BLK, 16), lambda i: (i, 0)),
                  _full((1, IN_CH)), _full((IN_CH, OUT)),
                  _full((16, IN_CH)), _full((OUT, 16)), _full((OUT, 16))],
        out_specs=[pl.BlockSpec((_BLK, OUT), lambda i: (i, 0)),
                   pl.BlockSpec((_BLK, 16), lambda i: (i, 0)),
                   pl.BlockSpec((_BLK, 16), lambda i: (i, 0)),
                   _full((8, 128))],
        out_shape=[jax.ShapeDtypeStruct((N, OUT), f32),
                   jax.ShapeDtypeStruct((N, 16), f32),
                   jax.ShapeDtypeStruct((N, 16), f32),
                   jax.ShapeDtypeStruct((8, 128), f32)],
    )(outm[0, :N], outm[1, :N], outw[0, :N], outw[1, :N],
      bias1[None, :], W2, R, AS2, AD2)

    ad2p_p = jnp.concatenate([ad2p, jnp.zeros((NPAD - N, 16), f32)], axis=0)
    m2, w2 = _make_sc2()(g2, as2p, ad2p_p, s_full, d_full, maxb2, ZW)

    out = pl.pallas_call(
        _final_body,
        grid=(_GRID,),
        in_specs=[pl.BlockSpec((_BLK, 16), lambda i: (i, 0)),
                  pl.BlockSpec((_BLK, 16), lambda i: (i, 0)),
                  pl.BlockSpec((_BLK, 16), lambda i: (i, 0)),
                  pl.BlockSpec((_BLK, 16), lambda i: (i, 0)),
                  _full((1, OUT)), _full((16, 16))],
        out_specs=[pl.BlockSpec((_BLK, OUT), lambda i: (i, 0))],
        out_shape=[jax.ShapeDtypeStruct((N, OUT), f32)],
    )(m2[0, :N], m2[1, :N], w2[0, :N], w2[1, :N], bias2[None, :], BSEL)

    return out[0]
